# Initial kernel scaffold; baseline (speedup 1.0000x reference)
import jax
import jax.numpy as jnp
from jax.experimental import pallas as pl


def kernel(x, tables):
    raise NotImplementedError("write your pallas kernel here")


# speedup vs baseline: 1.2042x; 1.2042x over previous
import functools

import jax
import jax.numpy as jnp
from jax import lax
from jax.experimental import pallas as pl
from jax.experimental.pallas import tpu as pltpu
from jax.experimental.pallas import tpu_sc as plsc

NUM_FIELDS = 26
VOCAB = 100000
EMB = 32
BATCH = 16384

NC, NS, L = 2, 16, 16
NW = NC * NS
ROWS = BATCH * NUM_FIELDS
ROWS_PER_W = ROWS // NW
CHUNK = 26 * 64
NCHUNK = ROWS_PER_W // CHUNK

_mesh = plsc.VectorSubcoreMesh(core_axis_name="c", subcore_axis_name="s")


@functools.partial(
    pl.kernel,
    out_type=jax.ShapeDtypeStruct((ROWS, EMB), jnp.float32),
    mesh=_mesh,
    compiler_params=pltpu.CompilerParams(use_tc_tiling_on_sc=False),
    scratch_types=[
        pltpu.VMEM((CHUNK,), jnp.int32),
        pltpu.VMEM((CHUNK,), jnp.int32),
        pltpu.VMEM((CHUNK, EMB), jnp.float32),
        pltpu.SemaphoreType.DMA,
    ],
)
def _gather_kernel(xflat_hbm, offs_hbm, table_hbm, out_hbm,
                   idx_v, offs_v, rows_v, sem):
    wid = lax.axis_index("s") * NC + lax.axis_index("c")
    pltpu.sync_copy(offs_hbm, offs_v)

    def chunk_body(ci, carry):
        base = wid * ROWS_PER_W + ci * CHUNK
        pltpu.sync_copy(xflat_hbm.at[pl.ds(base, CHUNK)], idx_v)

        def add_body(v, c):
            sl = pl.ds(v * L, L)
            idx_v[sl] = idx_v[sl] + offs_v[sl]
            return c

        lax.fori_loop(0, CHUNK // L, add_body, 0)
        pltpu.async_copy(table_hbm.at[idx_v], rows_v, sem).wait()
        pltpu.sync_copy(rows_v, out_hbm.at[pl.ds(base, CHUNK)])
        return carry

    lax.fori_loop(0, NCHUNK, chunk_body, 0)


def kernel(x, tables):
    if x.ndim <= 1:
        x = x[None, :]
    xflat = x.reshape(-1).astype(jnp.int32)
    offs = jnp.tile(
        jnp.arange(NUM_FIELDS, dtype=jnp.int32) * VOCAB, CHUNK // NUM_FIELDS
    )
    table = tables.reshape(NUM_FIELDS * VOCAB, EMB)
    out = _gather_kernel(xflat, offs, table)
    return out.reshape(BATCH, NUM_FIELDS * EMB)


# speedup vs baseline: 3.5507x; 2.9487x over previous
import functools

import jax
import jax.numpy as jnp
from jax import lax
from jax.experimental import pallas as pl
from jax.experimental.pallas import tpu as pltpu
from jax.experimental.pallas import tpu_sc as plsc

NUM_FIELDS = 26
VOCAB = 100000
EMB = 32
BATCH = 16384

NC, NS, L = 2, 16, 16
NW = NC * NS
P = NUM_FIELDS * EMB
OCHUNK = 4096

_mesh = plsc.VectorSubcoreMesh(core_axis_name="c", subcore_axis_name="s")


@functools.partial(
    pl.kernel,
    out_type=jax.ShapeDtypeStruct((P, BATCH), jnp.float32),
    mesh=_mesh,
    compiler_params=pltpu.CompilerParams(
        use_tc_tiling_on_sc=True, needs_layout_passes=False
    ),
    scratch_types=[
        pltpu.VMEM((VOCAB,), jnp.float32),
        pltpu.VMEM((BATCH,), jnp.int32),
        pltpu.VMEM((OCHUNK,), jnp.float32),
        pltpu.SemaphoreType.DMA,
    ],
)
def _gather_kernel(xt_hbm, tt_hbm, out_hbm, row_v, idx_v, out_v, sem):
    e = lax.axis_index("s") * NC + lax.axis_index("c")

    def field_body(f, carry):
        p = f * EMB + e
        pltpu.sync_copy(xt_hbm.at[f], idx_v)
        pltpu.sync_copy(tt_hbm.at[p], row_v)

        def chunk_body(c, carry2):
            def vec_body(j, carry3):
                sl = pl.ds(c * OCHUNK + j * L, L)
                out_v[pl.ds(j * L, L)] = plsc.load_gather(row_v, [idx_v[sl]])
                return carry3

            lax.fori_loop(0, OCHUNK // L, vec_body, 0)
            pltpu.sync_copy(out_v, out_hbm.at[p, pl.ds(c * OCHUNK, OCHUNK)])
            return carry2

        lax.fori_loop(0, BATCH // OCHUNK, chunk_body, 0)
        return carry

    lax.fori_loop(0, NUM_FIELDS, field_body, 0)


def kernel(x, tables):
    if x.ndim <= 1:
        x = x[None, :]
    xt = x.T.astype(jnp.int32)
    tt = tables.transpose(0, 2, 1).reshape(P, VOCAB)
    out_t = _gather_kernel(xt, tt)
    return (
        out_t.reshape(NUM_FIELDS, EMB, BATCH)
        .transpose(2, 0, 1)
        .reshape(BATCH, P)
    )


# speedup vs baseline: 3.5654x; 1.0041x over previous
import functools

import jax
import jax.numpy as jnp
from jax import lax
from jax.experimental import pallas as pl
from jax.experimental.pallas import tpu as pltpu
from jax.experimental.pallas import tpu_sc as plsc

NUM_FIELDS = 26
VOCAB = 100000
EMB = 32
BATCH = 16384

NC, NS, L = 2, 16, 16
NW = NC * NS
P = NUM_FIELDS * EMB
OCHUNK = 4096

_mesh = plsc.VectorSubcoreMesh(core_axis_name="c", subcore_axis_name="s")


@functools.partial(
    pl.kernel,
    out_type=jax.ShapeDtypeStruct((P, BATCH), jnp.float32),
    mesh=_mesh,
    compiler_params=pltpu.CompilerParams(
        use_tc_tiling_on_sc=True, needs_layout_passes=False
    ),
    scratch_types=[
        pltpu.VMEM((VOCAB,), jnp.float32),
        pltpu.VMEM((BATCH,), jnp.int32),
        pltpu.VMEM((2, OCHUNK), jnp.float32),
        pltpu.SemaphoreType.DMA,
        pltpu.SemaphoreType.DMA,
    ],
)
def _gather_kernel(xt_hbm, tt_hbm, out_hbm, row_v, idx_v, out_v, isem, osem):
    e = lax.axis_index("s") * NC + lax.axis_index("c")
    NCH = BATCH // OCHUNK
    UNROLL = 8

    def field_body(f, carry):
        p = f * EMB + e
        in1 = pltpu.async_copy(xt_hbm.at[f], idx_v, isem)
        in2 = pltpu.async_copy(tt_hbm.at[p], row_v, isem)
        in1.wait()
        in2.wait()

        outcps = []
        for c in range(NCH):
            b = c % 2
            if c >= 2:
                outcps[c - 2].wait()

            def vec_body(j, carry3):
                for u in range(UNROLL):
                    sl = pl.ds(c * OCHUNK + (j * UNROLL + u) * L, L)
                    out_v[b, pl.ds((j * UNROLL + u) * L, L)] = (
                        plsc.load_gather(row_v, [idx_v[sl]])
                    )
                return carry3

            lax.fori_loop(0, OCHUNK // (L * UNROLL), vec_body, 0)
            outcps.append(
                pltpu.async_copy(
                    out_v.at[b], out_hbm.at[p, pl.ds(c * OCHUNK, OCHUNK)], osem
                )
            )
        outcps[NCH - 2].wait()
        outcps[NCH - 1].wait()
        return carry

    lax.fori_loop(0, NUM_FIELDS, field_body, 0)


def kernel(x, tables):
    if x.ndim <= 1:
        x = x[None, :]
    xt = x.T.astype(jnp.int32)
    tt = tables.transpose(0, 2, 1).reshape(P, VOCAB)
    out_t = _gather_kernel(xt, tt)
    return (
        out_t.reshape(NUM_FIELDS, EMB, BATCH)
        .transpose(2, 0, 1)
        .reshape(BATCH, P)
    )


# speedup vs baseline: 6.1172x; 1.7157x over previous
import functools

import jax
import jax.numpy as jnp
from jax import lax
from jax.experimental import pallas as pl
from jax.experimental.pallas import tpu as pltpu
from jax.experimental.pallas import tpu_sc as plsc

NUM_FIELDS = 26
VOCAB = 100000
EMB = 32
BATCH = 16384

NC, NS, L = 2, 16, 16
NW = NC * NS
P = NUM_FIELDS * EMB
OCHUNK = 4096

_mesh = plsc.VectorSubcoreMesh(core_axis_name="c", subcore_axis_name="s")


@functools.partial(
    pl.kernel,
    out_type=jax.ShapeDtypeStruct((P, BATCH), jnp.float32),
    mesh=_mesh,
    compiler_params=pltpu.CompilerParams(
        use_tc_tiling_on_sc=True, needs_layout_passes=False
    ),
    scratch_types=[
        pltpu.VMEM((VOCAB,), jnp.float32),
        pltpu.VMEM((BATCH,), jnp.int32),
        pltpu.VMEM((2, OCHUNK), jnp.float32),
        pltpu.SemaphoreType.DMA,
        pltpu.SemaphoreType.DMA,
    ],
)
def _gather_kernel(xt_hbm, tt_hbm, out_hbm, row_v, idx_v, out_v, isem, osem):
    e = lax.axis_index("s") * NC + lax.axis_index("c")
    NCH = BATCH // OCHUNK
    UNROLL = 8

    def field_body(f, carry):
        p = f * EMB + e
        in1 = pltpu.async_copy(xt_hbm.at[f], idx_v, isem)
        in2 = pltpu.async_copy(tt_hbm.at[p], row_v, isem)
        in1.wait()
        in2.wait()

        outcps = []
        for c in range(NCH):
            b = c % 2
            if c >= 2:
                outcps[c - 2].wait()

            @plsc.parallel_loop(0, OCHUNK, step=L, unroll=UNROLL)
            def _gather_chunk(j):
                out_v[b, pl.ds(j, L)] = plsc.load_gather(
                    row_v, [idx_v[pl.ds(c * OCHUNK + j, L)]]
                )
            outcps.append(
                pltpu.async_copy(
                    out_v.at[b], out_hbm.at[p, pl.ds(c * OCHUNK, OCHUNK)], osem
                )
            )
        outcps[NCH - 2].wait()
        outcps[NCH - 1].wait()
        return carry

    lax.fori_loop(0, NUM_FIELDS, field_body, 0)


def kernel(x, tables):
    if x.ndim <= 1:
        x = x[None, :]
    xt = x.T.astype(jnp.int32)
    tt = tables.transpose(0, 2, 1).reshape(P, VOCAB)
    out_t = _gather_kernel(xt, tt)
    return (
        out_t.reshape(NUM_FIELDS, EMB, BATCH)
        .transpose(2, 0, 1)
        .reshape(BATCH, P)
    )


# speedup vs baseline: 8.0402x; 1.3144x over previous
import functools

import jax
import jax.numpy as jnp
from jax import lax
from jax.experimental import pallas as pl
from jax.experimental.pallas import tpu as pltpu
from jax.experimental.pallas import tpu_sc as plsc

NUM_FIELDS = 26
VOCAB = 100000
EMB = 32
BATCH = 16384

NC, NS, L = 2, 16, 16
NW = NC * NS
P = NUM_FIELDS * EMB
OCHUNK = 4096

_mesh = plsc.VectorSubcoreMesh(core_axis_name="c", subcore_axis_name="s")


@functools.partial(
    pl.kernel,
    out_type=jax.ShapeDtypeStruct((P, BATCH), jnp.float32),
    mesh=_mesh,
    compiler_params=pltpu.CompilerParams(
        use_tc_tiling_on_sc=True, needs_layout_passes=False
    ),
    scratch_types=[
        pltpu.VMEM((VOCAB,), jnp.float32),
        pltpu.VMEM((BATCH,), jnp.int32),
        pltpu.VMEM((2, OCHUNK), jnp.float32),
        pltpu.SemaphoreType.DMA,
        pltpu.SemaphoreType.DMA,
    ],
)
def _gather_kernel(xt_hbm, tt_hbm, out_hbm, row_v, idx_v, out_v, isem, osem):
    wid = lax.axis_index("s") * NC + lax.axis_index("c")
    NCH = BATCH // OCHUNK
    UNROLL = 8
    PPW = P // NW

    p_lo = wid * PPW
    p_hi = p_lo + PPW
    f_lo = p_lo // EMB
    f_hi = (p_hi - 1) // EMB

    def field_body(f, carry):
        pltpu.sync_copy(xt_hbm.at[f], idx_v)
        q_lo = lax.max(p_lo, f * EMB)
        q_hi = lax.min(p_hi, (f + 1) * EMB)

        def row_body(p, carry2):
            pltpu.async_copy(tt_hbm.at[p], row_v, isem).wait()

            outcps = []
            for c in range(NCH):
                b = c % 2
                if c >= 2:
                    outcps[c - 2].wait()

                @plsc.parallel_loop(0, OCHUNK, step=L, unroll=UNROLL)
                def _gather_chunk(j):
                    out_v[b, pl.ds(j, L)] = plsc.load_gather(
                        row_v, [idx_v[pl.ds(c * OCHUNK + j, L)]]
                    )
                outcps.append(
                    pltpu.async_copy(
                        out_v.at[b],
                        out_hbm.at[p, pl.ds(c * OCHUNK, OCHUNK)],
                        osem,
                    )
                )
            outcps[NCH - 2].wait()
            outcps[NCH - 1].wait()
            return carry2

        lax.fori_loop(q_lo, q_hi, row_body, 0)
        return carry

    lax.fori_loop(f_lo, f_hi + 1, field_body, 0)


def kernel(x, tables):
    if x.ndim <= 1:
        x = x[None, :]
    xt = x.T.astype(jnp.int32)
    tt = tables.transpose(0, 2, 1).reshape(P, VOCAB)
    out_t = _gather_kernel(xt, tt)
    return (
        out_t.reshape(NUM_FIELDS, EMB, BATCH)
        .transpose(2, 0, 1)
        .reshape(BATCH, P)
    )


# speedup vs baseline: 8.0482x; 1.0010x over previous
import functools

import jax
import jax.numpy as jnp
from jax import lax
from jax.experimental import pallas as pl
from jax.experimental.pallas import tpu as pltpu
from jax.experimental.pallas import tpu_sc as plsc

NUM_FIELDS = 26
VOCAB = 100000
EMB = 32
BATCH = 16384

NC, NS, L = 2, 16, 16
NW = NC * NS
P = NUM_FIELDS * EMB
OCHUNK = 4096

_mesh = plsc.VectorSubcoreMesh(core_axis_name="c", subcore_axis_name="s")


@functools.partial(
    pl.kernel,
    out_type=jax.ShapeDtypeStruct((P, BATCH), jnp.float32),
    mesh=_mesh,
    compiler_params=pltpu.CompilerParams(
        use_tc_tiling_on_sc=True, needs_layout_passes=False
    ),
    scratch_types=[
        pltpu.VMEM((VOCAB,), jnp.float32),
        pltpu.VMEM((BATCH,), jnp.int32),
        pltpu.VMEM((2, OCHUNK), jnp.float32),
        pltpu.SemaphoreType.DMA,
        pltpu.SemaphoreType.DMA,
        pltpu.SemaphoreType.DMA,
    ],
)
def _gather_kernel(xt_hbm, tt_hbm, out_hbm, row_v, idx_v, out_v,
                   isem, osem0, osem1):
    wid = lax.axis_index("s") * NC + lax.axis_index("c")
    NCH = BATCH // OCHUNK
    UNROLL = 8
    PPW = P // NW

    p_lo = wid * PPW
    p_hi = p_lo + PPW
    f_lo = p_lo // EMB
    f_hi = (p_hi - 1) // EMB

    def field_body(f, carry):
        pltpu.sync_copy(xt_hbm.at[f], idx_v)
        q_lo = lax.max(p_lo, f * EMB)
        q_hi = lax.min(p_hi, (f + 1) * EMB)

        def row_body(p, carry2):
            pltpu.async_copy(tt_hbm.at[p], row_v, isem).wait()

            osems = (osem0, osem1)
            prev = [None, None]
            for c in range(NCH):
                b = c % 2
                if prev[b] is not None:
                    prev[b].wait()

                @plsc.parallel_loop(0, OCHUNK, step=L, unroll=UNROLL)
                def _gather_chunk(j):
                    out_v[b, pl.ds(j, L)] = plsc.load_gather(
                        row_v, [idx_v[pl.ds(c * OCHUNK + j, L)]]
                    )
                prev[b] = pltpu.async_copy(
                    out_v.at[b],
                    out_hbm.at[p, pl.ds(c * OCHUNK, OCHUNK)],
                    osems[b],
                )
            prev[0].wait()
            prev[1].wait()
            return carry2

        lax.fori_loop(q_lo, q_hi, row_body, 0)
        return carry

    lax.fori_loop(f_lo, f_hi + 1, field_body, 0)


def kernel(x, tables):
    if x.ndim <= 1:
        x = x[None, :]
    xt = x.T.astype(jnp.int32)
    tt = tables.transpose(0, 2, 1).reshape(P, VOCAB)
    out_t = _gather_kernel(xt, tt)
    return (
        out_t.reshape(NUM_FIELDS, EMB, BATCH)
        .transpose(2, 0, 1)
        .reshape(BATCH, P)
    )
